# Initial kernel scaffold; baseline (speedup 1.0000x reference)
import jax
import jax.numpy as jnp
from jax.experimental import pallas as pl


def kernel(preds, tgt, tgt_pos, reward):
    raise NotImplementedError("write your pallas kernel here")


# speedup vs baseline: 1.6778x; 1.6778x over previous
import jax
import jax.numpy as jnp
from jax.experimental import pallas as pl
from jax.experimental.pallas import tpu as pltpu


def kernel(preds, tgt, tgt_pos, reward):
    del tgt_pos
    B, S, V = preds.shape
    N = B * S
    RB = 8
    assert N % RB == 0
    x = preds.reshape(N, V)
    t2 = tgt.reshape(N, 1).astype(jnp.int32)
    r2 = reward.reshape(N, 1)

    def body(x_ref, t_ref, r_ref, o_ref, acc_ref):
        i = pl.program_id(0)

        @pl.when(i == 0)
        def _init():
            acc_ref[0] = 0.0
            acc_ref[1] = 0.0

        xb = x_ref[...]
        tb = t_ref[...]
        m = jnp.max(xb, axis=1, keepdims=True)
        s = jnp.sum(jnp.exp(xb - m), axis=1, keepdims=True)
        col = jax.lax.broadcasted_iota(jnp.int32, (RB, V), 1)
        g = jnp.sum(jnp.where(col == tb, xb, 0.0), axis=1, keepdims=True)
        logp = g - m - jnp.log(s)
        valid = (tb > 0).astype(jnp.float32)
        acc_ref[0] += jnp.sum(logp * r_ref[...] * valid)
        acc_ref[1] += jnp.sum(valid)

        @pl.when(i == pl.num_programs(0) - 1)
        def _fin():
            o_ref[0, 0] = -(acc_ref[0] / jnp.maximum(acc_ref[1], 1.0))

    out = pl.pallas_call(
        body,
        grid=(N // RB,),
        in_specs=[
            pl.BlockSpec((RB, V), lambda i: (i, 0)),
            pl.BlockSpec((RB, 1), lambda i: (i, 0)),
            pl.BlockSpec((RB, 1), lambda i: (i, 0)),
        ],
        out_specs=pl.BlockSpec(memory_space=pltpu.SMEM),
        out_shape=jax.ShapeDtypeStruct((1, 1), jnp.float32),
        scratch_shapes=[pltpu.SMEM((2,), jnp.float32)],
    )(x, t2, r2)
    return out[0, 0]
